# baseline (speedup 1.0000x reference)
import functools

import jax
import jax.numpy as jnp
from jax import lax
from jax.experimental import pallas as pl
from jax.experimental.pallas import tpu as pltpu
from jax.experimental.pallas import tpu_sc as plsc

N_NODES = 50000
IN = 3
HID = 64
NL = 50
NC = 10

N_PAD = 50176
HALF = N_PAD // 2
ROWS_PER_TILE = HALF // 16
E_PAD_TILE = 50176
E_PAD = 16 * E_PAD_TILE
CONV_CHUNK = 1792
NPAIR = CONV_CHUNK // 256
NCHUNK = E_PAD_TILE // CONV_CHUNK
DEG_TILE = E_PAD // 32
DEG_CHUNK = 12544
HIST_W = N_PAD + 16

_mesh = plsc.VectorSubcoreMesh(core_axis_name="c", subcore_axis_name="s")


def _deg_body(dst_hbm, hists_hbm, hist, chunk):
    c = lax.axis_index("c")
    s = lax.axis_index("s")
    w = c * 16 + s
    base = w * DEG_TILE

    def zero_body(i, _):
        hist[pl.ds(i * 16, 16)] = jnp.zeros((16,), jnp.float32)
        return 0
    lax.fori_loop(0, HIST_W // 16, zero_body, 0)
    ones = jnp.ones((16,), jnp.float32)

    for k in range(DEG_TILE // DEG_CHUNK):
        pltpu.sync_copy(dst_hbm.at[pl.ds(base + k * DEG_CHUNK, DEG_CHUNK)], chunk)

        def blk(i, _):
            for v in range(8):
                idx = chunk[pl.ds(i * 128 + v * 16, 16)]
                plsc.addupdate_scatter(hist, [idx], ones)
            return 0
        lax.fori_loop(0, DEG_CHUNK // 128, blk, 0)

    pltpu.sync_copy(hist.at[pl.ds(0, N_PAD)], hists_hbm.at[w])


_sc_params = pltpu.CompilerParams(
    needs_layout_passes=False, use_tc_tiling_on_sc=False)

_deg_kernel = functools.partial(
    pl.kernel,
    out_type=jax.ShapeDtypeStruct((32, N_PAD), jnp.float32),
    mesh=_mesh,
    compiler_params=_sc_params,
    scratch_types=[
        pltpu.VMEM((HIST_W,), jnp.float32),
        pltpu.VMEM((DEG_CHUNK,), jnp.int32),
    ],
)(_deg_body)


def _conv_body(hs_hbm, src_hbm, dst_hbm, out_hbm, acc, srcb0, dstb0,
               srcb1, dstb1, idx0, idx1, gix0, gix1, rows0, rows1,
               gs0, gs1, ss0, ss1, ts0, ts1):
    c = lax.axis_index("c")
    s = lax.axis_index("s")
    lo = c * HALF
    srcb = [srcb0, srcb1]
    dstb = [dstb0, dstb1]
    idxb = [idx0, idx1]
    gixb = [gix0, gix1]
    rows = [rows0, rows1]
    gsem = [gs0, gs1]
    ssem = [ss0, ss1]
    tsem = [ts0, ts1]

    def zb(i, _):
        for j in range(4):
            rows0[i, pl.ds(j * 16, 16)] = jnp.zeros((16,), jnp.float32)
        return 0
    lax.fori_loop(0, 128, zb, 0)
    row0 = s * ROWS_PER_TILE
    for k in range(ROWS_PER_TILE // 128):
        pltpu.sync_copy(rows0, acc.at[pl.ds(row0 + k * 128, 128)])
    rem = ROWS_PER_TILE % 128
    if rem:
        pltpu.sync_copy(rows0.at[pl.ds(0, rem)],
                        acc.at[pl.ds(row0 + (ROWS_PER_TILE // 128) * 128, rem)])

    @pl.when(s == 0)
    def _():
        pltpu.sync_copy(rows0.at[pl.ds(0, 8)], acc.at[pl.ds(HALF, 8)])

    plsc.subcore_barrier()

    base_e = s * E_PAD_TILE

    def fire_g(b):
        pltpu.async_copy(hs_hbm.at[gixb[b]], rows[b], gsem[b])

    def wait_g(b):
        pltpu.make_async_copy(hs_hbm.at[pl.ds(0, 128)], rows[b],
                              gsem[b]).wait()

    def fire_s(b):
        pltpu.async_copy(rows[b], acc.at[idxb[b]], ssem[b], add=True)

    def wait_s(b):
        pltpu.make_async_copy(rows[b], acc.at[pl.ds(0, 128)], ssem[b]).wait()

    def build_idx(b, cs, cd, off):
        for v in range(8):
            dl = cd[pl.ds(off + v * 16, 16)] - lo
            inr = (dl >= 0) & (dl < HALF)
            idxb[b][pl.ds(v * 16, 16)] = jnp.where(inr, dl, HALF)
            sv = cs[pl.ds(off + v * 16, 16)]
            gixb[b][pl.ds(v * 16, 16)] = jnp.where(inr, sv, 0)

    def stage(k, p):
        off = base_e + k * CONV_CHUNK
        pltpu.async_copy(src_hbm.at[pl.ds(off, CONV_CHUNK)], srcb[p], tsem[p])
        pltpu.async_copy(dst_hbm.at[pl.ds(off, CONV_CHUNK)], dstb[p], tsem[p])

    def wait_stage(p):
        pltpu.make_async_copy(src_hbm.at[pl.ds(0, CONV_CHUNK)], srcb[p],
                              tsem[p]).wait()
        pltpu.make_async_copy(src_hbm.at[pl.ds(0, CONV_CHUNK)], dstb[p],
                              tsem[p]).wait()

    stage(0, 0)
    wait_stage(0)
    stage(1, 1)
    build_idx(0, srcb[0], dstb[0], 0)
    fire_g(0)
    build_idx(1, srcb[0], dstb[0], 128)
    fire_g(1)

    def chunk_pair(m, _):
        for p in range(2):
            k = 2 * m + p
            cs, cd = srcb[p], dstb[p]

            def body(j, _, cs=cs, cd=cd):
                wait_g(0)
                fire_s(0)
                wait_g(1)
                fire_s(1)
                wait_s(0)
                build_idx(0, cs, cd, j * 256 + 256)
                fire_g(0)
                wait_s(1)
                build_idx(1, cs, cd, j * 256 + 384)
                fire_g(1)
                return 0
            lax.fori_loop(0, NPAIR - 1, body, 0)
            wait_g(0)
            fire_s(0)
            wait_g(1)
            fire_s(1)

            @pl.when(k + 2 < NCHUNK)
            def _():
                stage(k + 2, p)

            @pl.when(k + 1 < NCHUNK)
            def _():
                wait_stage(1 - p)
                ncs, ncd = srcb[1 - p], dstb[1 - p]
                wait_s(0)
                build_idx(0, ncs, ncd, 0)
                fire_g(0)
                wait_s(1)
                build_idx(1, ncs, ncd, 128)
                fire_g(1)

            @pl.when(k + 1 >= NCHUNK)
            def _():
                wait_s(0)
                wait_s(1)
        return 0

    lax.fori_loop(0, NCHUNK // 2, chunk_pair, 0)

    plsc.subcore_barrier()
    pltpu.sync_copy(acc.at[pl.ds(row0, ROWS_PER_TILE)],
                    out_hbm.at[pl.ds(lo + row0, ROWS_PER_TILE)])


_conv_kernel = functools.partial(
    pl.kernel,
    out_type=jax.ShapeDtypeStruct((N_PAD, HID), jnp.float32),
    mesh=_mesh,
    compiler_params=_sc_params,
    scratch_types=[
        pltpu.VMEM_SHARED((HALF + 8, HID), jnp.float32),
        pltpu.VMEM((CONV_CHUNK,), jnp.int32),
        pltpu.VMEM((CONV_CHUNK,), jnp.int32),
        pltpu.VMEM((CONV_CHUNK,), jnp.int32),
        pltpu.VMEM((CONV_CHUNK,), jnp.int32),
        pltpu.VMEM((128,), jnp.int32),
        pltpu.VMEM((128,), jnp.int32),
        pltpu.VMEM((128,), jnp.int32),
        pltpu.VMEM((128,), jnp.int32),
        pltpu.VMEM((128, HID), jnp.float32),
        pltpu.VMEM((128, HID), jnp.float32),
        pltpu.SemaphoreType.DMA,
        pltpu.SemaphoreType.DMA,
        pltpu.SemaphoreType.DMA,
        pltpu.SemaphoreType.DMA,
        pltpu.SemaphoreType.DMA,
        pltpu.SemaphoreType.DMA,
    ],
)(_conv_body)


def _tc_b(x_ref, hists_ref, W1_ref, hs_ref, dinv_ref):
    deg = jnp.sum(hists_ref[...], axis=0) + 1.0
    dinv = lax.rsqrt(deg)[:, None]
    hs_ref[...] = (x_ref[...] @ W1_ref[...]) * dinv
    dinv_ref[...] = dinv


def _stage_b(x_pad, hists, W1):
    B = 3584
    return pl.pallas_call(
        _tc_b,
        grid=(N_PAD // B,),
        in_specs=[
            pl.BlockSpec((B, IN), lambda i: (i, 0)),
            pl.BlockSpec((32, B), lambda i: (0, i)),
            pl.BlockSpec((IN, HID), lambda i: (0, 0)),
        ],
        out_specs=[
            pl.BlockSpec((B, HID), lambda i: (i, 0)),
            pl.BlockSpec((B, 1), lambda i: (i, 0)),
        ],
        out_shape=[
            jax.ShapeDtypeStruct((N_PAD, HID), jnp.float32),
            jax.ShapeDtypeStruct((N_PAD, 1), jnp.float32),
        ],
    )(x_pad, hists, W1)


def _tc_d(acc_ref, hs_ref, dinv_ref, b_ref, W_ref, out_ref):
    dinv = dinv_ref[...]
    h = jax.nn.relu((acc_ref[...] + hs_ref[...]) * dinv + b_ref[...])
    out_ref[...] = (h @ W_ref[...]) * dinv


def _stage_d(acc, hs, dinv, b, W):
    B = 3584
    return pl.pallas_call(
        _tc_d,
        grid=(N_PAD // B,),
        in_specs=[
            pl.BlockSpec((B, HID), lambda i: (i, 0)),
            pl.BlockSpec((B, HID), lambda i: (i, 0)),
            pl.BlockSpec((B, 1), lambda i: (i, 0)),
            pl.BlockSpec((1, HID), lambda i: (0, 0)),
            pl.BlockSpec((HID, HID), lambda i: (0, 0)),
        ],
        out_specs=pl.BlockSpec((B, HID), lambda i: (i, 0)),
        out_shape=jax.ShapeDtypeStruct((N_PAD, HID), jnp.float32),
    )(acc, hs, dinv, b, W)


def _tc_f(acc_ref, hs_ref, dinv_ref, b2_ref, Wm_ref, bm_ref, fr_ref):
    h = jax.nn.relu((acc_ref[...] + hs_ref[...]) * dinv_ref[...] + b2_ref[...])
    h = jax.nn.relu(h @ Wm_ref[...] + bm_ref[...])
    fr_ref[...] = jnp.stack(
        [jnp.mean(h[i * NL:(i + 1) * NL], axis=0) for i in range(40)])


def _stage_f(acc, hs, dinv, b2, Wm, bm):
    return pl.pallas_call(
        _tc_f,
        grid=(N_NODES // (40 * NL),),
        in_specs=[
            pl.BlockSpec((40 * NL, HID), lambda i: (i, 0)),
            pl.BlockSpec((40 * NL, HID), lambda i: (i, 0)),
            pl.BlockSpec((40 * NL, 1), lambda i: (i, 0)),
            pl.BlockSpec((1, HID), lambda i: (0, 0)),
            pl.BlockSpec((HID, HID), lambda i: (0, 0)),
            pl.BlockSpec((1, HID), lambda i: (0, 0)),
        ],
        out_specs=pl.BlockSpec((40, HID), lambda i: (i, 0)),
        out_shape=jax.ShapeDtypeStruct((N_NODES // NL, HID), jnp.float32),
    )(acc, hs, dinv, b2, Wm, bm)


def _tc_g(fr_ref, scale_ref, Wih_ref, bih_ref, Whh_ref, bhh_ref,
          Wc1_ref, bc1_ref, Wc2_ref, bc2_ref, out_ref, gir_s, giz_s, gin_s):
    T = N_NODES // NL
    fr = fr_ref[...] * scale_ref[0, 0]
    gi = fr @ Wih_ref[...] + bih_ref[...]
    gir_s[...] = gi[:, 0:HID]
    giz_s[...] = gi[:, HID:2 * HID]
    gin_s[...] = gi[:, 2 * HID:3 * HID]
    Whh = Whh_ref[...]
    Whr = Whh[:, 0:HID]
    Whz = Whh[:, HID:2 * HID]
    Whn = Whh[:, 2 * HID:3 * HID]
    bhh = bhh_ref[...]
    bhr = bhh[:, 0:HID]
    bhz = bhh[:, HID:2 * HID]
    bhn = bhh[:, 2 * HID:3 * HID]

    def one(t, h):
        r = jax.nn.sigmoid(gir_s[pl.ds(t, 1), :] + h @ Whr + bhr)
        z = jax.nn.sigmoid(giz_s[pl.ds(t, 1), :] + h @ Whz + bhz)
        n = jnp.tanh(gin_s[pl.ds(t, 1), :] + r * (h @ Whn + bhn))
        return n + z * (h - n)

    def step(i, h):
        return one(2 * i + 1, one(2 * i, h))

    h = lax.fori_loop(0, T // 2, step, jnp.zeros((1, HID), jnp.float32))
    out_ref[...] = (jax.nn.relu(h @ Wc1_ref[...] + bc1_ref[...])
                    @ Wc2_ref[...] + bc2_ref[...])


def _stage_g(fr, scale, Wih, bih, Whh, bhh, Wc1, bc1, Wc2, bc2):
    return pl.pallas_call(
        _tc_g,
        out_shape=jax.ShapeDtypeStruct((1, NC), jnp.float32),
        scratch_shapes=[pltpu.VMEM((N_NODES // NL, HID), jnp.float32)] * 3,
    )(fr, scale, Wih, bih, Whh, bhh, Wc1, bc1, Wc2, bc2)


def kernel(x, edge_index, batch, num_landmarks, W1, b1, W2, b2, Wm, bm,
           Wih, Whh, bih, bhh, Wc1, bc1, Wc2, bc2):
    E = edge_index.shape[1]
    src = jnp.concatenate(
        [edge_index[0], jnp.zeros((E_PAD - E,), jnp.int32)])
    dst = jnp.concatenate(
        [edge_index[1], jnp.full((E_PAD - E,), N_PAD, jnp.int32)])
    x_pad = jnp.pad(x, ((0, N_PAD - x.shape[0]), (0, 0)))

    hists = _deg_kernel(dst)
    hs1, dinv = _stage_b(x_pad, hists, W1)
    acc1 = _conv_kernel(hs1, src, dst)
    hs2 = _stage_d(acc1, hs1, dinv, b1.reshape(1, HID), W2)
    acc2 = _conv_kernel(hs2, src, dst)
    fr = _stage_f(acc2, hs2, dinv, b2.reshape(1, HID), Wm, bm.reshape(1, HID))
    scale = (jnp.asarray(num_landmarks).astype(jnp.float32) / NL).reshape(1, 1)
    logits = _stage_g(fr, scale, Wih, bih.reshape(1, 3 * HID), Whh,
                      bhh.reshape(1, 3 * HID), Wc1, bc1.reshape(1, HID // 2),
                      Wc2, bc2.reshape(1, NC))
    return logits


# speedup vs baseline: 18.8475x; 18.8475x over previous
import functools

import jax
import jax.numpy as jnp
from jax import lax
from jax.experimental import pallas as pl
from jax.experimental.pallas import tpu as pltpu
from jax.experimental.pallas import tpu_sc as plsc

N_NODES = 50000
IN = 3
HID = 64
NL = 50
NC = 10

N_PAD = 50176
HALF = N_PAD // 2
ROWS_PER_TILE = HALF // 16
E_PAD_TILE = 50176
E_PAD = 16 * E_PAD_TILE
CONV_CHUNK = 1792
NPAIR = CONV_CHUNK // 256
NCHUNK = E_PAD_TILE // CONV_CHUNK
DEG_TILE = E_PAD // 32
DEG_CHUNK = 12544
HIST_W = N_PAD + 16

_mesh = plsc.VectorSubcoreMesh(core_axis_name="c", subcore_axis_name="s")


def _deg_body(dst_hbm, hists_hbm, hist, chunk):
    c = lax.axis_index("c")
    s = lax.axis_index("s")
    w = c * 16 + s
    base = w * DEG_TILE

    def zero_body(i, _):
        hist[pl.ds(i * 16, 16)] = jnp.zeros((16,), jnp.float32)
        return 0
    lax.fori_loop(0, HIST_W // 16, zero_body, 0)
    ones = jnp.ones((16,), jnp.float32)

    for k in range(DEG_TILE // DEG_CHUNK):
        pltpu.sync_copy(dst_hbm.at[pl.ds(base + k * DEG_CHUNK, DEG_CHUNK)], chunk)

        def blk(i, _):
            for v in range(8):
                idx = chunk[pl.ds(i * 128 + v * 16, 16)]
                plsc.addupdate_scatter(hist, [idx], ones)
            return 0
        lax.fori_loop(0, DEG_CHUNK // 128, blk, 0)

    pltpu.sync_copy(hist.at[pl.ds(0, N_PAD)], hists_hbm.at[w])


_sc_params = pltpu.CompilerParams(
    needs_layout_passes=False, use_tc_tiling_on_sc=False)

_deg_kernel = functools.partial(
    pl.kernel,
    out_type=jax.ShapeDtypeStruct((32, N_PAD), jnp.float32),
    mesh=_mesh,
    compiler_params=_sc_params,
    scratch_types=[
        pltpu.VMEM((HIST_W,), jnp.float32),
        pltpu.VMEM((DEG_CHUNK,), jnp.int32),
    ],
)(_deg_body)


def _conv_body(hs_hbm, src_hbm, dst_hbm, out_hbm, acc, srcb0, dstb0,
               srcb1, dstb1, idx0, idx1, gix0, gix1, rows0, rows1,
               gs0, gs1, ss0, ss1, ts0, ts1):
    c = lax.axis_index("c")
    s = lax.axis_index("s")
    lo = c * HALF
    srcb = [srcb0, srcb1]
    dstb = [dstb0, dstb1]
    idxb = [idx0, idx1]
    gixb = [gix0, gix1]
    rows = [rows0, rows1]
    gsem = [gs0, gs1]
    ssem = [ss0, ss1]
    tsem = [ts0, ts1]

    def zb(i, _):
        for j in range(4):
            rows0[i, pl.ds(j * 16, 16)] = jnp.zeros((16,), jnp.float32)
        return 0
    lax.fori_loop(0, 128, zb, 0)
    row0 = s * ROWS_PER_TILE
    for k in range(ROWS_PER_TILE // 128):
        pltpu.sync_copy(rows0, acc.at[pl.ds(row0 + k * 128, 128)])
    rem = ROWS_PER_TILE % 128
    if rem:
        pltpu.sync_copy(rows0.at[pl.ds(0, rem)],
                        acc.at[pl.ds(row0 + (ROWS_PER_TILE // 128) * 128, rem)])

    @pl.when(s == 0)
    def _():
        pltpu.sync_copy(rows0.at[pl.ds(0, 8)], acc.at[pl.ds(HALF, 8)])

    plsc.subcore_barrier()

    base_e = s * E_PAD_TILE

    def fire_g(b):
        pltpu.async_copy(hs_hbm.at[gixb[b]], rows[b], gsem[b])

    def wait_g(b):
        pltpu.make_async_copy(hs_hbm.at[pl.ds(0, 128)], rows[b],
                              gsem[b]).wait()

    def fire_s(b):
        pltpu.async_copy(rows[b], acc.at[idxb[b]], ssem[b], add=True)

    def wait_s(b):
        pltpu.make_async_copy(rows[b], acc.at[pl.ds(0, 128)], ssem[b]).wait()

    def build_idx(b, cs, cd, off):
        for v in range(8):
            dl = cd[pl.ds(off + v * 16, 16)] - lo
            inr = (dl >= 0) & (dl < HALF)
            idxb[b][pl.ds(v * 16, 16)] = jnp.where(inr, dl, HALF)
            gixb[b][pl.ds(v * 16, 16)] = cs[pl.ds(off + v * 16, 16)]

    def stage(k, p):
        off = base_e + k * CONV_CHUNK
        pltpu.async_copy(src_hbm.at[pl.ds(off, CONV_CHUNK)], srcb[p], tsem[p])
        pltpu.async_copy(dst_hbm.at[pl.ds(off, CONV_CHUNK)], dstb[p], tsem[p])

    def wait_stage(p):
        pltpu.make_async_copy(src_hbm.at[pl.ds(0, CONV_CHUNK)], srcb[p],
                              tsem[p]).wait()
        pltpu.make_async_copy(src_hbm.at[pl.ds(0, CONV_CHUNK)], dstb[p],
                              tsem[p]).wait()

    stage(0, 0)
    wait_stage(0)
    stage(1, 1)
    build_idx(0, srcb[0], dstb[0], 0)
    fire_g(0)
    build_idx(1, srcb[0], dstb[0], 128)
    fire_g(1)

    def chunk_pair(m, _):
        for p in range(2):
            k = 2 * m + p
            cs, cd = srcb[p], dstb[p]

            def body(j, _, cs=cs, cd=cd):
                wait_g(0)
                fire_s(0)
                wait_g(1)
                fire_s(1)
                wait_s(0)
                build_idx(0, cs, cd, j * 256 + 256)
                fire_g(0)
                wait_s(1)
                build_idx(1, cs, cd, j * 256 + 384)
                fire_g(1)
                return 0
            lax.fori_loop(0, NPAIR - 1, body, 0)
            wait_g(0)
            fire_s(0)
            wait_g(1)
            fire_s(1)

            @pl.when(k + 2 < NCHUNK)
            def _():
                stage(k + 2, p)

            @pl.when(k + 1 < NCHUNK)
            def _():
                wait_stage(1 - p)
                ncs, ncd = srcb[1 - p], dstb[1 - p]
                wait_s(0)
                build_idx(0, ncs, ncd, 0)
                fire_g(0)
                wait_s(1)
                build_idx(1, ncs, ncd, 128)
                fire_g(1)

            @pl.when(k + 1 >= NCHUNK)
            def _():
                wait_s(0)
                wait_s(1)
        return 0

    lax.fori_loop(0, NCHUNK // 2, chunk_pair, 0)

    plsc.subcore_barrier()
    pltpu.sync_copy(acc.at[pl.ds(row0, ROWS_PER_TILE)],
                    out_hbm.at[pl.ds(lo + row0, ROWS_PER_TILE)])


_conv_kernel = functools.partial(
    pl.kernel,
    out_type=jax.ShapeDtypeStruct((N_PAD, HID), jnp.float32),
    mesh=_mesh,
    compiler_params=_sc_params,
    scratch_types=[
        pltpu.VMEM_SHARED((HALF + 8, HID), jnp.float32),
        pltpu.VMEM((CONV_CHUNK,), jnp.int32),
        pltpu.VMEM((CONV_CHUNK,), jnp.int32),
        pltpu.VMEM((CONV_CHUNK,), jnp.int32),
        pltpu.VMEM((CONV_CHUNK,), jnp.int32),
        pltpu.VMEM((128,), jnp.int32),
        pltpu.VMEM((128,), jnp.int32),
        pltpu.VMEM((128,), jnp.int32),
        pltpu.VMEM((128,), jnp.int32),
        pltpu.VMEM((128, HID), jnp.float32),
        pltpu.VMEM((128, HID), jnp.float32),
        pltpu.SemaphoreType.DMA,
        pltpu.SemaphoreType.DMA,
        pltpu.SemaphoreType.DMA,
        pltpu.SemaphoreType.DMA,
        pltpu.SemaphoreType.DMA,
        pltpu.SemaphoreType.DMA,
    ],
)(_conv_body)


def _tc_b(x_ref, hists_ref, W1_ref, hs_ref, dinv_ref):
    deg = jnp.sum(hists_ref[...], axis=0) + 1.0
    dinv = lax.rsqrt(deg)[:, None]
    hs_ref[...] = (x_ref[...] @ W1_ref[...]) * dinv
    dinv_ref[...] = dinv


def _stage_b(x_pad, hists, W1):
    B = 3584
    return pl.pallas_call(
        _tc_b,
        grid=(N_PAD // B,),
        in_specs=[
            pl.BlockSpec((B, IN), lambda i: (i, 0)),
            pl.BlockSpec((32, B), lambda i: (0, i)),
            pl.BlockSpec((IN, HID), lambda i: (0, 0)),
        ],
        out_specs=[
            pl.BlockSpec((B, HID), lambda i: (i, 0)),
            pl.BlockSpec((B, 1), lambda i: (i, 0)),
        ],
        out_shape=[
            jax.ShapeDtypeStruct((N_PAD, HID), jnp.float32),
            jax.ShapeDtypeStruct((N_PAD, 1), jnp.float32),
        ],
    )(x_pad, hists, W1)


def _tc_d(acc_ref, hs_ref, dinv_ref, b_ref, W_ref, out_ref):
    dinv = dinv_ref[...]
    h = jax.nn.relu((acc_ref[...] + hs_ref[...]) * dinv + b_ref[...])
    out_ref[...] = (h @ W_ref[...]) * dinv


def _stage_d(acc, hs, dinv, b, W):
    B = 3584
    return pl.pallas_call(
        _tc_d,
        grid=(N_PAD // B,),
        in_specs=[
            pl.BlockSpec((B, HID), lambda i: (i, 0)),
            pl.BlockSpec((B, HID), lambda i: (i, 0)),
            pl.BlockSpec((B, 1), lambda i: (i, 0)),
            pl.BlockSpec((1, HID), lambda i: (0, 0)),
            pl.BlockSpec((HID, HID), lambda i: (0, 0)),
        ],
        out_specs=pl.BlockSpec((B, HID), lambda i: (i, 0)),
        out_shape=jax.ShapeDtypeStruct((N_PAD, HID), jnp.float32),
    )(acc, hs, dinv, b, W)


def _tc_f(acc_ref, hs_ref, dinv_ref, b2_ref, Wm_ref, bm_ref, fr_ref):
    h = jax.nn.relu((acc_ref[...] + hs_ref[...]) * dinv_ref[...] + b2_ref[...])
    h = jax.nn.relu(h @ Wm_ref[...] + bm_ref[...])
    fr_ref[...] = jnp.stack(
        [jnp.mean(h[i * NL:(i + 1) * NL], axis=0) for i in range(40)])


def _stage_f(acc, hs, dinv, b2, Wm, bm):
    return pl.pallas_call(
        _tc_f,
        grid=(N_NODES // (40 * NL),),
        in_specs=[
            pl.BlockSpec((40 * NL, HID), lambda i: (i, 0)),
            pl.BlockSpec((40 * NL, HID), lambda i: (i, 0)),
            pl.BlockSpec((40 * NL, 1), lambda i: (i, 0)),
            pl.BlockSpec((1, HID), lambda i: (0, 0)),
            pl.BlockSpec((HID, HID), lambda i: (0, 0)),
            pl.BlockSpec((1, HID), lambda i: (0, 0)),
        ],
        out_specs=pl.BlockSpec((40, HID), lambda i: (i, 0)),
        out_shape=jax.ShapeDtypeStruct((N_NODES // NL, HID), jnp.float32),
    )(acc, hs, dinv, b2, Wm, bm)


def _tc_g(fr_ref, scale_ref, Wih_ref, bih_ref, Whh_ref, bhh_ref,
          Wc1_ref, bc1_ref, Wc2_ref, bc2_ref, out_ref, gir_s, giz_s, gin_s):
    T = N_NODES // NL
    fr = fr_ref[...] * scale_ref[0, 0]
    gi = fr @ Wih_ref[...] + bih_ref[...]
    gir_s[...] = gi[:, 0:HID]
    giz_s[...] = gi[:, HID:2 * HID]
    gin_s[...] = gi[:, 2 * HID:3 * HID]
    Whh = Whh_ref[...]
    Whr = Whh[:, 0:HID]
    Whz = Whh[:, HID:2 * HID]
    Whn = Whh[:, 2 * HID:3 * HID]
    bhh = bhh_ref[...]
    bhr = bhh[:, 0:HID]
    bhz = bhh[:, HID:2 * HID]
    bhn = bhh[:, 2 * HID:3 * HID]

    def one(t, h):
        r = jax.nn.sigmoid(gir_s[pl.ds(t, 1), :] + h @ Whr + bhr)
        z = jax.nn.sigmoid(giz_s[pl.ds(t, 1), :] + h @ Whz + bhz)
        n = jnp.tanh(gin_s[pl.ds(t, 1), :] + r * (h @ Whn + bhn))
        return n + z * (h - n)

    def step(i, h):
        return one(2 * i + 1, one(2 * i, h))

    h = lax.fori_loop(0, T // 2, step, jnp.zeros((1, HID), jnp.float32))
    out_ref[...] = (jax.nn.relu(h @ Wc1_ref[...] + bc1_ref[...])
                    @ Wc2_ref[...] + bc2_ref[...])


def _stage_g(fr, scale, Wih, bih, Whh, bhh, Wc1, bc1, Wc2, bc2):
    return pl.pallas_call(
        _tc_g,
        out_shape=jax.ShapeDtypeStruct((1, NC), jnp.float32),
        scratch_shapes=[pltpu.VMEM((N_NODES // NL, HID), jnp.float32)] * 3,
    )(fr, scale, Wih, bih, Whh, bhh, Wc1, bc1, Wc2, bc2)


def kernel(x, edge_index, batch, num_landmarks, W1, b1, W2, b2, Wm, bm,
           Wih, Whh, bih, bhh, Wc1, bc1, Wc2, bc2):
    E = edge_index.shape[1]
    src = jnp.concatenate(
        [edge_index[0], jnp.zeros((E_PAD - E,), jnp.int32)])
    dst = jnp.concatenate(
        [edge_index[1], jnp.full((E_PAD - E,), N_PAD, jnp.int32)])
    x_pad = jnp.pad(x, ((0, N_PAD - x.shape[0]), (0, 0)))

    hists = _deg_kernel(dst)
    hs1, dinv = _stage_b(x_pad, hists, W1)
    acc1 = _conv_kernel(hs1, src, dst)
    hs2 = _stage_d(acc1, hs1, dinv, b1.reshape(1, HID), W2)
    acc2 = _conv_kernel(hs2, src, dst)
    fr = _stage_f(acc2, hs2, dinv, b2.reshape(1, HID), Wm, bm.reshape(1, HID))
    scale = (jnp.asarray(num_landmarks).astype(jnp.float32) / NL).reshape(1, 1)
    logits = _stage_g(fr, scale, Wih, bih.reshape(1, 3 * HID), Whh,
                      bhh.reshape(1, 3 * HID), Wc1, bc1.reshape(1, HID // 2),
                      Wc2, bc2.reshape(1, NC))
    return logits


# speedup vs baseline: 19.8248x; 1.0519x over previous
import functools

import jax
import jax.numpy as jnp
from jax import lax
from jax.experimental import pallas as pl
from jax.experimental.pallas import tpu as pltpu
from jax.experimental.pallas import tpu_sc as plsc

N_NODES = 50000
IN = 3
HID = 64
NL = 50
NC = 10

N_PAD = 50176
HALF = N_PAD // 2
ROWS_PER_TILE = HALF // 16
E_PAD_TILE = 50176
E_PAD = 16 * E_PAD_TILE
CONV_CHUNK = 1792
NPAIR = CONV_CHUNK // 256
NCHUNK = E_PAD_TILE // CONV_CHUNK
DEG_TILE = E_PAD // 32
DEG_CHUNK = 12544
HIST_W = N_PAD + 16

_mesh = plsc.VectorSubcoreMesh(core_axis_name="c", subcore_axis_name="s")


DEGR = 512


def _deg_body(dst_hbm, deg_hbm, hist, chunk, zb, idc, degacc):
    c = lax.axis_index("c")
    s = lax.axis_index("s")
    w = c * 16 + s
    base = w * DEG_TILE

    def zero_body(i, _):
        for j in range(8):
            hist[i, pl.ds(j * 16, 16)] = jnp.zeros((16,), jnp.float32)
        return 0
    lax.fori_loop(0, DEGR, zero_body, 0)
    ones = jnp.ones((16,), jnp.float32)

    for k in range(DEG_TILE // DEG_CHUNK):
        pltpu.sync_copy(dst_hbm.at[pl.ds(base + k * DEG_CHUNK, DEG_CHUNK)], chunk)

        def blk(i, _):
            for v in range(8):
                idx = chunk[pl.ds(i * 128 + v * 16, 16)]
                plsc.addupdate_scatter(hist, [idx >> 7, idx & 127], ones)
            return 0
        lax.fori_loop(0, DEG_CHUNK // 128, blk, 0)

    @pl.when(s == 0)
    def _():
        def zzb(i, _):
            for j in range(8):
                zb[i, pl.ds(j * 16, 16)] = jnp.zeros((16,), jnp.float32)
            return 0
        lax.fori_loop(0, 128, zzb, 0)
        for r in range(DEGR // 128):
            pltpu.sync_copy(zb, degacc.at[pl.ds(r * 128, 128)])

    plsc.subcore_barrier()
    for r in range(DEGR // 128):
        for v in range(8):
            idc[pl.ds(v * 16, 16)] = (lax.iota(jnp.int32, 16)
                                      + (r * 128 + v * 16))
        pltpu.sync_copy(hist.at[pl.ds(r * 128, 128)], degacc.at[idc],
                        add=True)
    plsc.subcore_barrier()
    pltpu.sync_copy(degacc.at[pl.ds(s * (DEGR // 16), DEGR // 16)],
                    deg_hbm.at[c, pl.ds(s * (DEGR // 16), DEGR // 16)])


_sc_params = pltpu.CompilerParams(
    needs_layout_passes=False, use_tc_tiling_on_sc=False)

_deg_kernel = functools.partial(
    pl.kernel,
    out_type=jax.ShapeDtypeStruct((2, DEGR, 128), jnp.float32),
    mesh=_mesh,
    compiler_params=_sc_params,
    scratch_types=[
        pltpu.VMEM((DEGR, 128), jnp.float32),
        pltpu.VMEM((DEG_CHUNK,), jnp.int32),
        pltpu.VMEM((128, 128), jnp.float32),
        pltpu.VMEM((128,), jnp.int32),
        pltpu.VMEM_SHARED((DEGR, 128), jnp.float32),
    ],
)(_deg_body)


def _conv_body(hs_hbm, src_hbm, dst_hbm, out_hbm, acc, srcb0, dstb0,
               srcb1, dstb1, idx0, idx1, gix0, gix1, rows0, rows1,
               gs0, gs1, ss0, ss1, ts0, ts1):
    c = lax.axis_index("c")
    s = lax.axis_index("s")
    lo = c * HALF
    srcb = [srcb0, srcb1]
    dstb = [dstb0, dstb1]
    idxb = [idx0, idx1]
    gixb = [gix0, gix1]
    rows = [rows0, rows1]
    gsem = [gs0, gs1]
    ssem = [ss0, ss1]
    tsem = [ts0, ts1]

    def zb(i, _):
        for j in range(4):
            rows0[i, pl.ds(j * 16, 16)] = jnp.zeros((16,), jnp.float32)
        return 0
    lax.fori_loop(0, 128, zb, 0)
    row0 = s * ROWS_PER_TILE
    for k in range(ROWS_PER_TILE // 128):
        pltpu.sync_copy(rows0, acc.at[pl.ds(row0 + k * 128, 128)])
    rem = ROWS_PER_TILE % 128
    if rem:
        pltpu.sync_copy(rows0.at[pl.ds(0, rem)],
                        acc.at[pl.ds(row0 + (ROWS_PER_TILE // 128) * 128, rem)])

    @pl.when(s == 0)
    def _():
        pltpu.sync_copy(rows0.at[pl.ds(0, 8)], acc.at[pl.ds(HALF, 8)])

    plsc.subcore_barrier()

    base_e = s * E_PAD_TILE

    def fire_g(b):
        pltpu.async_copy(hs_hbm.at[gixb[b]], rows[b], gsem[b])

    def wait_g(b):
        pltpu.make_async_copy(hs_hbm.at[pl.ds(0, 128)], rows[b],
                              gsem[b]).wait()

    def fire_s(b):
        pltpu.async_copy(rows[b], acc.at[idxb[b]], ssem[b], add=True)

    def wait_s(b):
        pltpu.make_async_copy(rows[b], acc.at[pl.ds(0, 128)], ssem[b]).wait()

    def build_idx(b, cs, cd, off):
        for v in range(8):
            dl = cd[pl.ds(off + v * 16, 16)] - lo
            inr = (dl >= 0) & (dl < HALF)
            idxb[b][pl.ds(v * 16, 16)] = jnp.where(inr, dl, HALF)
            gixb[b][pl.ds(v * 16, 16)] = cs[pl.ds(off + v * 16, 16)]

    def stage(k, p):
        off = base_e + k * CONV_CHUNK
        pltpu.async_copy(src_hbm.at[pl.ds(off, CONV_CHUNK)], srcb[p], tsem[p])
        pltpu.async_copy(dst_hbm.at[pl.ds(off, CONV_CHUNK)], dstb[p], tsem[p])

    def wait_stage(p):
        pltpu.make_async_copy(src_hbm.at[pl.ds(0, CONV_CHUNK)], srcb[p],
                              tsem[p]).wait()
        pltpu.make_async_copy(src_hbm.at[pl.ds(0, CONV_CHUNK)], dstb[p],
                              tsem[p]).wait()

    stage(0, 0)
    wait_stage(0)
    stage(1, 1)
    build_idx(0, srcb[0], dstb[0], 0)
    fire_g(0)
    build_idx(1, srcb[0], dstb[0], 128)
    fire_g(1)

    def chunk_pair(m, _):
        for p in range(2):
            k = 2 * m + p
            cs, cd = srcb[p], dstb[p]

            def body(j, _, cs=cs, cd=cd):
                wait_g(0)
                fire_s(0)
                wait_g(1)
                fire_s(1)
                wait_s(0)
                build_idx(0, cs, cd, j * 256 + 256)
                fire_g(0)
                wait_s(1)
                build_idx(1, cs, cd, j * 256 + 384)
                fire_g(1)
                return 0
            lax.fori_loop(0, NPAIR - 1, body, 0)
            wait_g(0)
            fire_s(0)
            wait_g(1)
            fire_s(1)

            @pl.when(k + 2 < NCHUNK)
            def _():
                stage(k + 2, p)

            @pl.when(k + 1 < NCHUNK)
            def _():
                wait_stage(1 - p)
                ncs, ncd = srcb[1 - p], dstb[1 - p]
                wait_s(0)
                build_idx(0, ncs, ncd, 0)
                fire_g(0)
                wait_s(1)
                build_idx(1, ncs, ncd, 128)
                fire_g(1)

            @pl.when(k + 1 >= NCHUNK)
            def _():
                wait_s(0)
                wait_s(1)
        return 0

    lax.fori_loop(0, NCHUNK // 2, chunk_pair, 0)

    plsc.subcore_barrier()
    pltpu.sync_copy(acc.at[pl.ds(row0, ROWS_PER_TILE)],
                    out_hbm.at[pl.ds(lo + row0, ROWS_PER_TILE)])


_conv_kernel = functools.partial(
    pl.kernel,
    out_type=jax.ShapeDtypeStruct((N_PAD, HID), jnp.float32),
    mesh=_mesh,
    compiler_params=_sc_params,
    scratch_types=[
        pltpu.VMEM_SHARED((HALF + 8, HID), jnp.float32),
        pltpu.VMEM((CONV_CHUNK,), jnp.int32),
        pltpu.VMEM((CONV_CHUNK,), jnp.int32),
        pltpu.VMEM((CONV_CHUNK,), jnp.int32),
        pltpu.VMEM((CONV_CHUNK,), jnp.int32),
        pltpu.VMEM((128,), jnp.int32),
        pltpu.VMEM((128,), jnp.int32),
        pltpu.VMEM((128,), jnp.int32),
        pltpu.VMEM((128,), jnp.int32),
        pltpu.VMEM((128, HID), jnp.float32),
        pltpu.VMEM((128, HID), jnp.float32),
        pltpu.SemaphoreType.DMA,
        pltpu.SemaphoreType.DMA,
        pltpu.SemaphoreType.DMA,
        pltpu.SemaphoreType.DMA,
        pltpu.SemaphoreType.DMA,
        pltpu.SemaphoreType.DMA,
    ],
)(_conv_body)


NP2 = N_PAD // 2


def _tc_b(x_ref, deg_ref, W1p_ref, E_ref, hs_ref):
    dinv_p = lax.rsqrt(deg_ref[...] + 1.0) @ E_ref[...]
    hs_ref[...] = (x_ref[...] @ W1p_ref[...]) * dinv_p


def _stage_b(x_p, deg2, W1p, E):
    B = 1792
    return pl.pallas_call(
        _tc_b,
        grid=(NP2 // B,),
        in_specs=[
            pl.BlockSpec((B, 2 * IN), lambda i: (i, 0)),
            pl.BlockSpec((B, 2), lambda i: (i, 0)),
            pl.BlockSpec((2 * IN, 2 * HID), lambda i: (0, 0)),
            pl.BlockSpec((2, 2 * HID), lambda i: (0, 0)),
        ],
        out_specs=pl.BlockSpec((B, 2 * HID), lambda i: (i, 0)),
        out_shape=jax.ShapeDtypeStruct((NP2, 2 * HID), jnp.float32),
    )(x_p, deg2, W1p, E)


def _tc_d(acc_ref, hs_ref, deg_ref, E_ref, b_ref, W_ref, out_ref):
    dinv_p = lax.rsqrt(deg_ref[...] + 1.0) @ E_ref[...]
    h = jax.nn.relu((acc_ref[...] + hs_ref[...]) * dinv_p + b_ref[...])
    out_ref[...] = (h @ W_ref[...]) * dinv_p


def _stage_d(acc_p, hs_p, deg2, E, b_p, W_p):
    B = 1792
    return pl.pallas_call(
        _tc_d,
        grid=(NP2 // B,),
        in_specs=[
            pl.BlockSpec((B, 2 * HID), lambda i: (i, 0)),
            pl.BlockSpec((B, 2 * HID), lambda i: (i, 0)),
            pl.BlockSpec((B, 2), lambda i: (i, 0)),
            pl.BlockSpec((2, 2 * HID), lambda i: (0, 0)),
            pl.BlockSpec((1, 2 * HID), lambda i: (0, 0)),
            pl.BlockSpec((2 * HID, 2 * HID), lambda i: (0, 0)),
        ],
        out_specs=pl.BlockSpec((B, 2 * HID), lambda i: (i, 0)),
        out_shape=jax.ShapeDtypeStruct((NP2, 2 * HID), jnp.float32),
    )(acc_p, hs_p, deg2, E, b_p, W_p)


def _tc_f(acc_ref, hs_ref, deg_ref, E_ref, b2_ref, Wm_ref, bm_ref, fr_ref):
    dinv_p = lax.rsqrt(deg_ref[...] + 1.0) @ E_ref[...]
    h = jax.nn.relu((acc_ref[...] + hs_ref[...]) * dinv_p + b2_ref[...])
    h = jax.nn.relu(h @ Wm_ref[...] + bm_ref[...])
    S = jnp.stack(
        [jnp.sum(h[i * (NL // 2):(i + 1) * (NL // 2)], axis=0)
         for i in range(40)])
    fr_ref[...] = (S[:, 0:HID] + S[:, HID:2 * HID]) * (1.0 / NL)


def _stage_f(acc_p, hs_p, deg2, E, b2_p, Wm_p, bm_p):
    B = 1000
    return pl.pallas_call(
        _tc_f,
        grid=(N_NODES // (2 * B),),
        in_specs=[
            pl.BlockSpec((B, 2 * HID), lambda i: (i, 0)),
            pl.BlockSpec((B, 2 * HID), lambda i: (i, 0)),
            pl.BlockSpec((B, 2), lambda i: (i, 0)),
            pl.BlockSpec((2, 2 * HID), lambda i: (0, 0)),
            pl.BlockSpec((1, 2 * HID), lambda i: (0, 0)),
            pl.BlockSpec((2 * HID, 2 * HID), lambda i: (0, 0)),
            pl.BlockSpec((1, 2 * HID), lambda i: (0, 0)),
        ],
        out_specs=pl.BlockSpec((40, HID), lambda i: (i, 0)),
        out_shape=jax.ShapeDtypeStruct((N_NODES // NL, HID), jnp.float32),
    )(acc_p, hs_p, deg2, E, b2_p, Wm_p, bm_p)


def _tc_g(fr_ref, scale_ref, Wih_ref, bih_ref, Whh_ref, bhh_ref,
          Wc1_ref, bc1_ref, Wc2_ref, bc2_ref, out_ref, gir_s, giz_s, gin_s):
    T = N_NODES // NL
    fr = fr_ref[...] * scale_ref[0, 0]
    gi = fr @ Wih_ref[...] + bih_ref[...]
    gir_s[...] = gi[:, 0:HID]
    giz_s[...] = gi[:, HID:2 * HID]
    gin_s[...] = gi[:, 2 * HID:3 * HID]
    Whh = Whh_ref[...]
    Whr = Whh[:, 0:HID]
    Whz = Whh[:, HID:2 * HID]
    Whn = Whh[:, 2 * HID:3 * HID]
    bhh = bhh_ref[...]
    bhr = bhh[:, 0:HID]
    bhz = bhh[:, HID:2 * HID]
    bhn = bhh[:, 2 * HID:3 * HID]

    def one(t, h):
        r = jax.nn.sigmoid(gir_s[pl.ds(t, 1), :] + h @ Whr + bhr)
        z = jax.nn.sigmoid(giz_s[pl.ds(t, 1), :] + h @ Whz + bhz)
        n = jnp.tanh(gin_s[pl.ds(t, 1), :] + r * (h @ Whn + bhn))
        return n + z * (h - n)

    def step(i, h):
        return one(2 * i + 1, one(2 * i, h))

    h = lax.fori_loop(0, T // 2, step, jnp.zeros((1, HID), jnp.float32))
    out_ref[...] = (jax.nn.relu(h @ Wc1_ref[...] + bc1_ref[...])
                    @ Wc2_ref[...] + bc2_ref[...])


def _stage_g(fr, scale, Wih, bih, Whh, bhh, Wc1, bc1, Wc2, bc2):
    return pl.pallas_call(
        _tc_g,
        out_shape=jax.ShapeDtypeStruct((1, NC), jnp.float32),
        scratch_shapes=[pltpu.VMEM((N_NODES // NL, HID), jnp.float32)] * 3,
    )(fr, scale, Wih, bih, Whh, bhh, Wc1, bc1, Wc2, bc2)


def kernel(x, edge_index, batch, num_landmarks, W1, b1, W2, b2, Wm, bm,
           Wih, Whh, bih, bhh, Wc1, bc1, Wc2, bc2):
    E = edge_index.shape[1]
    src = jnp.concatenate(
        [edge_index[0], jnp.zeros((E_PAD - E,), jnp.int32)])
    dst = jnp.concatenate(
        [edge_index[1], jnp.full((E_PAD - E,), N_PAD, jnp.int32)])
    x_pad = jnp.pad(x, ((0, N_PAD - x.shape[0]), (0, 0)))

    Z = jnp.zeros((HID, HID), jnp.float32)
    Zi = jnp.zeros((IN, HID), jnp.float32)
    W1p = jnp.concatenate([jnp.concatenate([W1, Zi], 1),
                           jnp.concatenate([Zi, W1], 1)], 0)
    W2p = jnp.concatenate([jnp.concatenate([W2, Z], 1),
                           jnp.concatenate([Z, W2], 1)], 0)
    Wmp = jnp.concatenate([jnp.concatenate([Wm, Z], 1),
                           jnp.concatenate([Z, Wm], 1)], 0)
    E = jnp.repeat(jnp.eye(2, dtype=jnp.float32), HID, axis=1)
    b1p = jnp.concatenate([b1, b1]).reshape(1, 2 * HID)
    b2p = jnp.concatenate([b2, b2]).reshape(1, 2 * HID)
    bmp = jnp.concatenate([bm, bm]).reshape(1, 2 * HID)

    x_p = x_pad.reshape(NP2, 2 * IN)
    degparts = _deg_kernel(dst)
    deg2 = (degparts[0] + degparts[1]).reshape(-1)[:N_PAD].reshape(NP2, 2)
    hs1_p = _stage_b(x_p, deg2, W1p, E)
    acc1 = _conv_kernel(hs1_p.reshape(N_PAD, HID), src, dst)
    hs2_p = _stage_d(acc1.reshape(NP2, 2 * HID), hs1_p, deg2, E, b1p, W2p)
    acc2 = _conv_kernel(hs2_p.reshape(N_PAD, HID), src, dst)
    fr = _stage_f(acc2.reshape(NP2, 2 * HID), hs2_p, deg2, E, b2p, Wmp, bmp)
    scale = (jnp.asarray(num_landmarks).astype(jnp.float32) / NL).reshape(1, 1)
    logits = _stage_g(fr, scale, Wih, bih.reshape(1, 3 * HID), Whh,
                      bhh.reshape(1, 3 * HID), Wc1, bc1.reshape(1, HID // 2),
                      Wc2, bc2.reshape(1, NC))
    return logits


# speedup vs baseline: 20.0441x; 1.0111x over previous
import functools

import jax
import jax.numpy as jnp
from jax import lax
from jax.experimental import pallas as pl
from jax.experimental.pallas import tpu as pltpu
from jax.experimental.pallas import tpu_sc as plsc

N_NODES = 50000
IN = 3
HID = 64
NL = 50
NC = 10

N_PAD = 50176
HALF = N_PAD // 2
ROWS_PER_TILE = HALF // 16
E_PAD_TILE = 50176
E_PAD = 16 * E_PAD_TILE
CONV_CHUNK = 1792
NPAIR = CONV_CHUNK // 256
NCHUNK = E_PAD_TILE // CONV_CHUNK
DEG_TILE = E_PAD // 32
DEG_CHUNK = 12544
HIST_W = N_PAD + 16

_mesh = plsc.VectorSubcoreMesh(core_axis_name="c", subcore_axis_name="s")


DEGR = 512


def _deg_body(dst_hbm, deg_hbm, hist, chunk, zb, idc, degacc):
    c = lax.axis_index("c")
    s = lax.axis_index("s")
    w = c * 16 + s
    base = w * DEG_TILE

    def zero_body(i, _):
        for j in range(8):
            hist[i, pl.ds(j * 16, 16)] = jnp.zeros((16,), jnp.float32)
        return 0
    lax.fori_loop(0, DEGR, zero_body, 0)
    ones = jnp.ones((16,), jnp.float32)

    for k in range(DEG_TILE // DEG_CHUNK):
        pltpu.sync_copy(dst_hbm.at[pl.ds(base + k * DEG_CHUNK, DEG_CHUNK)], chunk)

        def blk(i, _):
            for v in range(8):
                idx = chunk[pl.ds(i * 128 + v * 16, 16)]
                plsc.addupdate_scatter(hist, [idx >> 7, idx & 127], ones)
            return 0
        lax.fori_loop(0, DEG_CHUNK // 128, blk, 0)

    @pl.when(s == 0)
    def _():
        def zzb(i, _):
            for j in range(8):
                zb[i, pl.ds(j * 16, 16)] = jnp.zeros((16,), jnp.float32)
            return 0
        lax.fori_loop(0, 128, zzb, 0)
        for r in range(DEGR // 128):
            pltpu.sync_copy(zb, degacc.at[pl.ds(r * 128, 128)])

    plsc.subcore_barrier()
    for r in range(DEGR // 128):
        for v in range(8):
            idc[pl.ds(v * 16, 16)] = (lax.iota(jnp.int32, 16)
                                      + (r * 128 + v * 16))
        pltpu.sync_copy(hist.at[pl.ds(r * 128, 128)], degacc.at[idc],
                        add=True)
    plsc.subcore_barrier()
    pltpu.sync_copy(degacc.at[pl.ds(s * (DEGR // 16), DEGR // 16)],
                    deg_hbm.at[c, pl.ds(s * (DEGR // 16), DEGR // 16)])


_sc_params = pltpu.CompilerParams(
    needs_layout_passes=False, use_tc_tiling_on_sc=False)

_deg_kernel = functools.partial(
    pl.kernel,
    out_type=jax.ShapeDtypeStruct((2, DEGR, 128), jnp.float32),
    mesh=_mesh,
    compiler_params=_sc_params,
    scratch_types=[
        pltpu.VMEM((DEGR, 128), jnp.float32),
        pltpu.VMEM((DEG_CHUNK,), jnp.int32),
        pltpu.VMEM((128, 128), jnp.float32),
        pltpu.VMEM((128,), jnp.int32),
        pltpu.VMEM_SHARED((DEGR, 128), jnp.float32),
    ],
)(_deg_body)


def _conv_body(hs_hbm, src_hbm, dst_hbm, out_hbm, acc, srcb0, dstb0,
               srcb1, dstb1, idx0, idx1, gix0, gix1, rows0, rows1,
               gs0, gs1, ss0, ss1, ts0, ts1):
    c = lax.axis_index("c")
    s = lax.axis_index("s")
    lo = c * HALF
    srcb = [srcb0, srcb1]
    dstb = [dstb0, dstb1]
    idxb = [idx0, idx1]
    gixb = [gix0, gix1]
    rows = [rows0, rows1]
    gsem = [gs0, gs1]
    ssem = [ss0, ss1]
    tsem = [ts0, ts1]

    def zb(i, _):
        for j in range(4):
            rows0[i, pl.ds(j * 16, 16)] = jnp.zeros((16,), jnp.float32)
        return 0
    lax.fori_loop(0, 128, zb, 0)
    row0 = s * ROWS_PER_TILE
    for k in range(ROWS_PER_TILE // 128):
        pltpu.sync_copy(rows0, acc.at[pl.ds(row0 + k * 128, 128)])
    rem = ROWS_PER_TILE % 128
    if rem:
        pltpu.sync_copy(rows0.at[pl.ds(0, rem)],
                        acc.at[pl.ds(row0 + (ROWS_PER_TILE // 128) * 128, rem)])

    @pl.when(s == 0)
    def _():
        pltpu.sync_copy(rows0.at[pl.ds(0, 8)], acc.at[pl.ds(HALF, 8)])

    plsc.subcore_barrier()

    base_e = s * E_PAD_TILE

    def fire_g(b):
        pltpu.async_copy(hs_hbm.at[gixb[b]], rows[b], gsem[b])

    def wait_g(b):
        pltpu.make_async_copy(hs_hbm.at[pl.ds(0, 128)], rows[b],
                              gsem[b]).wait()

    def fire_s(b):
        pltpu.async_copy(rows[b], acc.at[idxb[b]], ssem[b], add=True)

    def wait_s(b):
        pltpu.make_async_copy(rows[b], acc.at[pl.ds(0, 128)], ssem[b]).wait()

    def build_idx(b, cs, cd, off):
        for v in range(8):
            dl = cd[pl.ds(off + v * 16, 16)] - lo
            inr = (dl >= 0) & (dl < HALF)
            idxb[b][pl.ds(v * 16, 16)] = jnp.where(inr, dl, HALF)
            gixb[b][pl.ds(v * 16, 16)] = cs[pl.ds(off + v * 16, 16)]

    def stage(k, p):
        off = base_e + k * CONV_CHUNK
        pltpu.async_copy(src_hbm.at[pl.ds(off, CONV_CHUNK)], srcb[p], tsem[p])
        pltpu.async_copy(dst_hbm.at[pl.ds(off, CONV_CHUNK)], dstb[p], tsem[p])

    def wait_stage(p):
        pltpu.make_async_copy(src_hbm.at[pl.ds(0, CONV_CHUNK)], srcb[p],
                              tsem[p]).wait()
        pltpu.make_async_copy(src_hbm.at[pl.ds(0, CONV_CHUNK)], dstb[p],
                              tsem[p]).wait()

    stage(0, 0)
    wait_stage(0)
    stage(1, 1)
    build_idx(0, srcb[0], dstb[0], 0)
    fire_g(0)
    build_idx(1, srcb[0], dstb[0], 128)
    fire_g(1)

    def chunk_pair(m, _):
        for p in range(2):
            k = 2 * m + p
            cs, cd = srcb[p], dstb[p]

            def body(j, _, cs=cs, cd=cd):
                wait_g(0)
                fire_s(0)
                wait_g(1)
                fire_s(1)
                wait_s(0)
                build_idx(0, cs, cd, j * 256 + 256)
                fire_g(0)
                wait_s(1)
                build_idx(1, cs, cd, j * 256 + 384)
                fire_g(1)
                return 0
            lax.fori_loop(0, NPAIR - 1, body, 0)
            wait_g(0)
            fire_s(0)
            wait_g(1)
            fire_s(1)

            @pl.when(k + 2 < NCHUNK)
            def _():
                stage(k + 2, p)

            @pl.when(k + 1 < NCHUNK)
            def _():
                wait_stage(1 - p)
                ncs, ncd = srcb[1 - p], dstb[1 - p]
                wait_s(0)
                build_idx(0, ncs, ncd, 0)
                fire_g(0)
                wait_s(1)
                build_idx(1, ncs, ncd, 128)
                fire_g(1)

            @pl.when(k + 1 >= NCHUNK)
            def _():
                wait_s(0)
                wait_s(1)
        return 0

    lax.fori_loop(0, NCHUNK // 2, chunk_pair, 0)

    plsc.subcore_barrier()
    pltpu.sync_copy(acc.at[pl.ds(row0, ROWS_PER_TILE)],
                    out_hbm.at[pl.ds(lo + row0, ROWS_PER_TILE)])


_conv_kernel = functools.partial(
    pl.kernel,
    out_type=jax.ShapeDtypeStruct((N_PAD, HID), jnp.float32),
    mesh=_mesh,
    compiler_params=_sc_params,
    scratch_types=[
        pltpu.VMEM_SHARED((HALF + 8, HID), jnp.float32),
        pltpu.VMEM((CONV_CHUNK,), jnp.int32),
        pltpu.VMEM((CONV_CHUNK,), jnp.int32),
        pltpu.VMEM((CONV_CHUNK,), jnp.int32),
        pltpu.VMEM((CONV_CHUNK,), jnp.int32),
        pltpu.VMEM((128,), jnp.int32),
        pltpu.VMEM((128,), jnp.int32),
        pltpu.VMEM((128,), jnp.int32),
        pltpu.VMEM((128,), jnp.int32),
        pltpu.VMEM((128, HID), jnp.float32),
        pltpu.VMEM((128, HID), jnp.float32),
        pltpu.SemaphoreType.DMA,
        pltpu.SemaphoreType.DMA,
        pltpu.SemaphoreType.DMA,
        pltpu.SemaphoreType.DMA,
        pltpu.SemaphoreType.DMA,
        pltpu.SemaphoreType.DMA,
    ],
)(_conv_body)


NP2 = N_PAD // 2


def _tc_b(x_ref, deg_ref, W1p_ref, E_ref, hs_ref):
    dinv_p = lax.rsqrt(deg_ref[...] + 1.0) @ E_ref[...]
    hs_ref[...] = (x_ref[...] @ W1p_ref[...]) * dinv_p


def _stage_b(x_p, deg2, W1p, E):
    B = 1792
    return pl.pallas_call(
        _tc_b,
        grid=(NP2 // B,),
        in_specs=[
            pl.BlockSpec((B, 2 * IN), lambda i: (i, 0)),
            pl.BlockSpec((B, 2), lambda i: (i, 0)),
            pl.BlockSpec((2 * IN, 2 * HID), lambda i: (0, 0)),
            pl.BlockSpec((2, 2 * HID), lambda i: (0, 0)),
        ],
        out_specs=pl.BlockSpec((B, 2 * HID), lambda i: (i, 0)),
        out_shape=jax.ShapeDtypeStruct((NP2, 2 * HID), jnp.float32),
    )(x_p, deg2, W1p, E)


def _tc_d(acc_ref, hs_ref, deg_ref, E_ref, b_ref, W_ref, out_ref):
    dinv_p = lax.rsqrt(deg_ref[...] + 1.0) @ E_ref[...]
    h = jax.nn.relu((acc_ref[...] + hs_ref[...]) * dinv_p + b_ref[...])
    out_ref[...] = (h @ W_ref[...]) * dinv_p


def _stage_d(acc_p, hs_p, deg2, E, b_p, W_p):
    B = 1792
    return pl.pallas_call(
        _tc_d,
        grid=(NP2 // B,),
        in_specs=[
            pl.BlockSpec((B, 2 * HID), lambda i: (i, 0)),
            pl.BlockSpec((B, 2 * HID), lambda i: (i, 0)),
            pl.BlockSpec((B, 2), lambda i: (i, 0)),
            pl.BlockSpec((2, 2 * HID), lambda i: (0, 0)),
            pl.BlockSpec((1, 2 * HID), lambda i: (0, 0)),
            pl.BlockSpec((2 * HID, 2 * HID), lambda i: (0, 0)),
        ],
        out_specs=pl.BlockSpec((B, 2 * HID), lambda i: (i, 0)),
        out_shape=jax.ShapeDtypeStruct((NP2, 2 * HID), jnp.float32),
    )(acc_p, hs_p, deg2, E, b_p, W_p)


def _tc_f(acc_ref, hs_ref, deg_ref, E_ref, b2_ref, Wm_ref, bm_ref, fr_ref):
    dinv_p = lax.rsqrt(deg_ref[...] + 1.0) @ E_ref[...]
    h = jax.nn.relu((acc_ref[...] + hs_ref[...]) * dinv_p + b2_ref[...])
    h = jax.nn.relu(h @ Wm_ref[...] + bm_ref[...])
    S = jnp.stack(
        [jnp.sum(h[i * (NL // 2):(i + 1) * (NL // 2)], axis=0)
         for i in range(40)])
    fr_ref[...] = (S[:, 0:HID] + S[:, HID:2 * HID]) * (1.0 / NL)


def _stage_f(acc_p, hs_p, deg2, E, b2_p, Wm_p, bm_p):
    B = 1000
    return pl.pallas_call(
        _tc_f,
        grid=(N_NODES // (2 * B),),
        in_specs=[
            pl.BlockSpec((B, 2 * HID), lambda i: (i, 0)),
            pl.BlockSpec((B, 2 * HID), lambda i: (i, 0)),
            pl.BlockSpec((B, 2), lambda i: (i, 0)),
            pl.BlockSpec((2, 2 * HID), lambda i: (0, 0)),
            pl.BlockSpec((1, 2 * HID), lambda i: (0, 0)),
            pl.BlockSpec((2 * HID, 2 * HID), lambda i: (0, 0)),
            pl.BlockSpec((1, 2 * HID), lambda i: (0, 0)),
        ],
        out_specs=pl.BlockSpec((40, HID), lambda i: (i, 0)),
        out_shape=jax.ShapeDtypeStruct((N_NODES // NL, HID), jnp.float32),
    )(acc_p, hs_p, deg2, E, b2_p, Wm_p, bm_p)


def _tc_g(fr_ref, scale_ref, Wih_ref, bih_ref, Whh_ref, bhh_ref,
          Wc1_ref, bc1_ref, Wc2_ref, bc2_ref, out_ref, gir_s, giz_s, gin_s):
    T = N_NODES // NL
    fr = fr_ref[...] * scale_ref[0, 0]
    gi = fr @ Wih_ref[...] + bih_ref[...]
    gir_s[...] = gi[:, 0:HID]
    giz_s[...] = gi[:, HID:2 * HID]
    gin_s[...] = gi[:, 2 * HID:3 * HID]
    Whh = Whh_ref[...]
    Whr = Whh[:, 0:HID]
    Whz = Whh[:, HID:2 * HID]
    Whn = Whh[:, 2 * HID:3 * HID]
    bhh = bhh_ref[...]
    bhr = bhh[:, 0:HID]
    bhz = bhh[:, HID:2 * HID]
    bhn = bhh[:, 2 * HID:3 * HID]

    def one(t, h):
        r = jax.nn.sigmoid(gir_s[pl.ds(t, 1), :] + h @ Whr + bhr)
        z = jax.nn.sigmoid(giz_s[pl.ds(t, 1), :] + h @ Whz + bhz)
        n = jnp.tanh(gin_s[pl.ds(t, 1), :] + r * (h @ Whn + bhn))
        return n + z * (h - n)

    def step(i, h):
        for u in range(4):
            h = one(4 * i + u, h)
        return h

    h = lax.fori_loop(0, T // 4, step, jnp.zeros((1, HID), jnp.float32))
    out_ref[...] = (jax.nn.relu(h @ Wc1_ref[...] + bc1_ref[...])
                    @ Wc2_ref[...] + bc2_ref[...])


def _stage_g(fr, scale, Wih, bih, Whh, bhh, Wc1, bc1, Wc2, bc2):
    return pl.pallas_call(
        _tc_g,
        out_shape=jax.ShapeDtypeStruct((1, NC), jnp.float32),
        scratch_shapes=[pltpu.VMEM((N_NODES // NL, HID), jnp.float32)] * 3,
    )(fr, scale, Wih, bih, Whh, bhh, Wc1, bc1, Wc2, bc2)


def kernel(x, edge_index, batch, num_landmarks, W1, b1, W2, b2, Wm, bm,
           Wih, Whh, bih, bhh, Wc1, bc1, Wc2, bc2):
    E = edge_index.shape[1]
    src = jnp.concatenate(
        [edge_index[0], jnp.zeros((E_PAD - E,), jnp.int32)])
    dst = jnp.concatenate(
        [edge_index[1], jnp.full((E_PAD - E,), N_PAD, jnp.int32)])
    x_pad = jnp.pad(x, ((0, N_PAD - x.shape[0]), (0, 0)))

    Z = jnp.zeros((HID, HID), jnp.float32)
    Zi = jnp.zeros((IN, HID), jnp.float32)
    W1p = jnp.concatenate([jnp.concatenate([W1, Zi], 1),
                           jnp.concatenate([Zi, W1], 1)], 0)
    W2p = jnp.concatenate([jnp.concatenate([W2, Z], 1),
                           jnp.concatenate([Z, W2], 1)], 0)
    Wmp = jnp.concatenate([jnp.concatenate([Wm, Z], 1),
                           jnp.concatenate([Z, Wm], 1)], 0)
    E = jnp.repeat(jnp.eye(2, dtype=jnp.float32), HID, axis=1)
    b1p = jnp.concatenate([b1, b1]).reshape(1, 2 * HID)
    b2p = jnp.concatenate([b2, b2]).reshape(1, 2 * HID)
    bmp = jnp.concatenate([bm, bm]).reshape(1, 2 * HID)

    x_p = x_pad.reshape(NP2, 2 * IN)
    degparts = _deg_kernel(dst)
    deg2 = (degparts[0] + degparts[1]).reshape(-1)[:N_PAD].reshape(NP2, 2)
    hs1_p = _stage_b(x_p, deg2, W1p, E)
    acc1 = _conv_kernel(hs1_p.reshape(N_PAD, HID), src, dst)
    hs2_p = _stage_d(acc1.reshape(NP2, 2 * HID), hs1_p, deg2, E, b1p, W2p)
    acc2 = _conv_kernel(hs2_p.reshape(N_PAD, HID), src, dst)
    fr = _stage_f(acc2.reshape(NP2, 2 * HID), hs2_p, deg2, E, b2p, Wmp, bmp)
    scale = (jnp.asarray(num_landmarks).astype(jnp.float32) / NL).reshape(1, 1)
    logits = _stage_g(fr, scale, Wih, bih.reshape(1, 3 * HID), Whh,
                      bhh.reshape(1, 3 * HID), Wc1, bc1.reshape(1, HID // 2),
                      Wc2, bc2.reshape(1, NC))
    return logits


# speedup vs baseline: 20.0577x; 1.0007x over previous
import functools

import jax
import jax.numpy as jnp
from jax import lax
from jax.experimental import pallas as pl
from jax.experimental.pallas import tpu as pltpu
from jax.experimental.pallas import tpu_sc as plsc

N_NODES = 50000
IN = 3
HID = 64
NL = 50
NC = 10

N_PAD = 50176
HALF = N_PAD // 2
ROWS_PER_TILE = HALF // 16
E_PAD_TILE = 50176
E_PAD = 16 * E_PAD_TILE
CONV_CHUNK = 1792
NPAIR = CONV_CHUNK // 256
NCHUNK = E_PAD_TILE // CONV_CHUNK
DEG_TILE = E_PAD // 32
DEG_CHUNK = 12544
HIST_W = N_PAD + 16

_mesh = plsc.VectorSubcoreMesh(core_axis_name="c", subcore_axis_name="s")


DEGR = 512


def _deg_body(dst_hbm, deg_hbm, hist, chunk, zb, idc, degacc):
    c = lax.axis_index("c")
    s = lax.axis_index("s")
    w = c * 16 + s
    base = w * DEG_TILE

    def zero_body(i, _):
        for j in range(8):
            hist[i, pl.ds(j * 16, 16)] = jnp.zeros((16,), jnp.float32)
        return 0
    lax.fori_loop(0, DEGR, zero_body, 0)
    ones = jnp.ones((16,), jnp.float32)

    for k in range(DEG_TILE // DEG_CHUNK):
        pltpu.sync_copy(dst_hbm.at[pl.ds(base + k * DEG_CHUNK, DEG_CHUNK)], chunk)

        def blk(i, _):
            for v in range(8):
                idx = chunk[pl.ds(i * 128 + v * 16, 16)]
                plsc.addupdate_scatter(hist, [idx >> 7, idx & 127], ones)
            return 0
        lax.fori_loop(0, DEG_CHUNK // 128, blk, 0)

    @pl.when(s == 0)
    def _():
        def zzb(i, _):
            for j in range(8):
                zb[i, pl.ds(j * 16, 16)] = jnp.zeros((16,), jnp.float32)
            return 0
        lax.fori_loop(0, 128, zzb, 0)
        for r in range(DEGR // 128):
            pltpu.sync_copy(zb, degacc.at[pl.ds(r * 128, 128)])

    plsc.subcore_barrier()
    for r in range(DEGR // 128):
        for v in range(8):
            idc[pl.ds(v * 16, 16)] = (lax.iota(jnp.int32, 16)
                                      + (r * 128 + v * 16))
        pltpu.sync_copy(hist.at[pl.ds(r * 128, 128)], degacc.at[idc],
                        add=True)
    plsc.subcore_barrier()
    pltpu.sync_copy(degacc.at[pl.ds(s * (DEGR // 16), DEGR // 16)],
                    deg_hbm.at[c, pl.ds(s * (DEGR // 16), DEGR // 16)])


_sc_params = pltpu.CompilerParams(
    needs_layout_passes=False, use_tc_tiling_on_sc=False)

_deg_kernel = functools.partial(
    pl.kernel,
    out_type=jax.ShapeDtypeStruct((2, DEGR, 128), jnp.float32),
    mesh=_mesh,
    compiler_params=_sc_params,
    scratch_types=[
        pltpu.VMEM((DEGR, 128), jnp.float32),
        pltpu.VMEM((DEG_CHUNK,), jnp.int32),
        pltpu.VMEM((128, 128), jnp.float32),
        pltpu.VMEM((128,), jnp.int32),
        pltpu.VMEM_SHARED((DEGR, 128), jnp.float32),
    ],
)(_deg_body)


def _conv_body(hs_hbm, src_hbm, dst_hbm, out_hbm, acc, srcb0, dstb0,
               srcb1, dstb1, idx0, idx1, gix0, gix1, rows0, rows1,
               gs0, gs1, ss0, ss1, ts0, ts1):
    c = lax.axis_index("c")
    s = lax.axis_index("s")
    lo = c * HALF
    srcb = [srcb0, srcb1]
    dstb = [dstb0, dstb1]
    idxb = [idx0, idx1]
    gixb = [gix0, gix1]
    rows = [rows0, rows1]
    gsem = [gs0, gs1]
    ssem = [ss0, ss1]
    tsem = [ts0, ts1]

    def zb(i, _):
        for j in range(4):
            rows0[i, pl.ds(j * 16, 16)] = jnp.zeros((16,), jnp.float32)
        return 0
    lax.fori_loop(0, 128, zb, 0)
    row0 = s * ROWS_PER_TILE
    for k in range(ROWS_PER_TILE // 128):
        pltpu.sync_copy(rows0, acc.at[pl.ds(row0 + k * 128, 128)])
    rem = ROWS_PER_TILE % 128
    if rem:
        pltpu.sync_copy(rows0.at[pl.ds(0, rem)],
                        acc.at[pl.ds(row0 + (ROWS_PER_TILE // 128) * 128, rem)])

    @pl.when(s == 0)
    def _():
        pltpu.sync_copy(rows0.at[pl.ds(0, 8)], acc.at[pl.ds(HALF, 8)])

    plsc.subcore_barrier()

    base_e = s * E_PAD_TILE

    def fire_g(b):
        pltpu.async_copy(hs_hbm.at[gixb[b]], rows[b], gsem[b])

    def wait_g(b):
        pltpu.make_async_copy(hs_hbm.at[pl.ds(0, 128)], rows[b],
                              gsem[b]).wait()

    def fire_s(b):
        pltpu.async_copy(rows[b], acc.at[idxb[b]], ssem[b], add=True)

    def wait_s(b):
        pltpu.make_async_copy(rows[b], acc.at[pl.ds(0, 128)], ssem[b]).wait()

    def build_idx(b, cs, cd, off):
        for v in range(8):
            dl = cd[pl.ds(off + v * 16, 16)] - lo
            inr = (dl >= 0) & (dl < HALF)
            idxb[b][pl.ds(v * 16, 16)] = jnp.where(inr, dl, HALF)
            gixb[b][pl.ds(v * 16, 16)] = cs[pl.ds(off + v * 16, 16)]

    def stage(k, p):
        off = base_e + k * CONV_CHUNK
        pltpu.async_copy(src_hbm.at[pl.ds(off, CONV_CHUNK)], srcb[p], tsem[p])
        pltpu.async_copy(dst_hbm.at[pl.ds(off, CONV_CHUNK)], dstb[p], tsem[p])

    def wait_stage(p):
        pltpu.make_async_copy(src_hbm.at[pl.ds(0, CONV_CHUNK)], srcb[p],
                              tsem[p]).wait()
        pltpu.make_async_copy(src_hbm.at[pl.ds(0, CONV_CHUNK)], dstb[p],
                              tsem[p]).wait()

    stage(0, 0)
    wait_stage(0)
    stage(1, 1)
    build_idx(0, srcb[0], dstb[0], 0)
    fire_g(0)
    build_idx(1, srcb[0], dstb[0], 128)
    fire_g(1)

    def chunk_pair(m, _):
        for p in range(2):
            k = 2 * m + p
            cs, cd = srcb[p], dstb[p]

            def body(j, _, cs=cs, cd=cd):
                wait_g(0)
                fire_s(0)
                wait_g(1)
                fire_s(1)
                wait_s(0)
                build_idx(0, cs, cd, j * 256 + 256)
                fire_g(0)
                wait_s(1)
                build_idx(1, cs, cd, j * 256 + 384)
                fire_g(1)
                return 0
            lax.fori_loop(0, NPAIR - 1, body, 0)
            wait_g(0)
            fire_s(0)
            wait_g(1)
            fire_s(1)

            @pl.when(k + 2 < NCHUNK)
            def _():
                stage(k + 2, p)

            @pl.when(k + 1 < NCHUNK)
            def _():
                wait_stage(1 - p)
                ncs, ncd = srcb[1 - p], dstb[1 - p]
                wait_s(0)
                build_idx(0, ncs, ncd, 0)
                fire_g(0)
                wait_s(1)
                build_idx(1, ncs, ncd, 128)
                fire_g(1)

            @pl.when(k + 1 >= NCHUNK)
            def _():
                wait_s(0)
                wait_s(1)
        return 0

    lax.fori_loop(0, NCHUNK // 2, chunk_pair, 0)

    plsc.subcore_barrier()
    pltpu.sync_copy(acc.at[pl.ds(row0, ROWS_PER_TILE)],
                    out_hbm.at[pl.ds(lo + row0, ROWS_PER_TILE)])


_conv_kernel = functools.partial(
    pl.kernel,
    out_type=jax.ShapeDtypeStruct((N_PAD, HID), jnp.float32),
    mesh=_mesh,
    compiler_params=_sc_params,
    scratch_types=[
        pltpu.VMEM_SHARED((HALF + 8, HID), jnp.float32),
        pltpu.VMEM((CONV_CHUNK,), jnp.int32),
        pltpu.VMEM((CONV_CHUNK,), jnp.int32),
        pltpu.VMEM((CONV_CHUNK,), jnp.int32),
        pltpu.VMEM((CONV_CHUNK,), jnp.int32),
        pltpu.VMEM((128,), jnp.int32),
        pltpu.VMEM((128,), jnp.int32),
        pltpu.VMEM((128,), jnp.int32),
        pltpu.VMEM((128,), jnp.int32),
        pltpu.VMEM((128, HID), jnp.float32),
        pltpu.VMEM((128, HID), jnp.float32),
        pltpu.SemaphoreType.DMA,
        pltpu.SemaphoreType.DMA,
        pltpu.SemaphoreType.DMA,
        pltpu.SemaphoreType.DMA,
        pltpu.SemaphoreType.DMA,
        pltpu.SemaphoreType.DMA,
    ],
)(_conv_body)


NP2 = N_PAD // 2


def _tc_b(x_ref, deg_ref, W1p_ref, E_ref, hs_ref):
    dinv_p = lax.rsqrt(deg_ref[...] + 1.0) @ E_ref[...]
    hs_ref[...] = (x_ref[...] @ W1p_ref[...]) * dinv_p


def _stage_b(x_p, deg2, W1p, E):
    B = 1792
    return pl.pallas_call(
        _tc_b,
        grid=(NP2 // B,),
        in_specs=[
            pl.BlockSpec((B, 2 * IN), lambda i: (i, 0)),
            pl.BlockSpec((B, 2), lambda i: (i, 0)),
            pl.BlockSpec((2 * IN, 2 * HID), lambda i: (0, 0)),
            pl.BlockSpec((2, 2 * HID), lambda i: (0, 0)),
        ],
        out_specs=pl.BlockSpec((B, 2 * HID), lambda i: (i, 0)),
        out_shape=jax.ShapeDtypeStruct((NP2, 2 * HID), jnp.float32),
    )(x_p, deg2, W1p, E)


def _tc_d(acc_ref, hs_ref, deg_ref, E_ref, b_ref, W_ref, out_ref):
    dinv_p = lax.rsqrt(deg_ref[...] + 1.0) @ E_ref[...]
    h = jax.nn.relu((acc_ref[...] + hs_ref[...]) * dinv_p + b_ref[...])
    out_ref[...] = (h @ W_ref[...]) * dinv_p


def _stage_d(acc_p, hs_p, deg2, E, b_p, W_p):
    B = 1792
    return pl.pallas_call(
        _tc_d,
        grid=(NP2 // B,),
        in_specs=[
            pl.BlockSpec((B, 2 * HID), lambda i: (i, 0)),
            pl.BlockSpec((B, 2 * HID), lambda i: (i, 0)),
            pl.BlockSpec((B, 2), lambda i: (i, 0)),
            pl.BlockSpec((2, 2 * HID), lambda i: (0, 0)),
            pl.BlockSpec((1, 2 * HID), lambda i: (0, 0)),
            pl.BlockSpec((2 * HID, 2 * HID), lambda i: (0, 0)),
        ],
        out_specs=pl.BlockSpec((B, 2 * HID), lambda i: (i, 0)),
        out_shape=jax.ShapeDtypeStruct((NP2, 2 * HID), jnp.float32),
    )(acc_p, hs_p, deg2, E, b_p, W_p)


def _tc_f(acc_ref, hs_ref, deg_ref, E_ref, b2_ref, Wm_ref, bm_ref, fr_ref):
    dinv_p = lax.rsqrt(deg_ref[...] + 1.0) @ E_ref[...]
    h = jax.nn.relu((acc_ref[...] + hs_ref[...]) * dinv_p + b2_ref[...])
    h = jax.nn.relu(h @ Wm_ref[...] + bm_ref[...])
    S = jnp.stack(
        [jnp.sum(h[i * (NL // 2):(i + 1) * (NL // 2)], axis=0)
         for i in range(40)])
    fr_ref[...] = (S[:, 0:HID] + S[:, HID:2 * HID]) * (1.0 / NL)


def _stage_f(acc_p, hs_p, deg2, E, b2_p, Wm_p, bm_p):
    B = 1000
    return pl.pallas_call(
        _tc_f,
        grid=(N_NODES // (2 * B),),
        in_specs=[
            pl.BlockSpec((B, 2 * HID), lambda i: (i, 0)),
            pl.BlockSpec((B, 2 * HID), lambda i: (i, 0)),
            pl.BlockSpec((B, 2), lambda i: (i, 0)),
            pl.BlockSpec((2, 2 * HID), lambda i: (0, 0)),
            pl.BlockSpec((1, 2 * HID), lambda i: (0, 0)),
            pl.BlockSpec((2 * HID, 2 * HID), lambda i: (0, 0)),
            pl.BlockSpec((1, 2 * HID), lambda i: (0, 0)),
        ],
        out_specs=pl.BlockSpec((40, HID), lambda i: (i, 0)),
        out_shape=jax.ShapeDtypeStruct((N_NODES // NL, HID), jnp.float32),
    )(acc_p, hs_p, deg2, E, b2_p, Wm_p, bm_p)


def _tc_g(fr_ref, scale_ref, Wih_ref, bih_ref, Whh_ref, bhh_ref,
          Wc1_ref, bc1_ref, Wc2_ref, bc2_ref, out_ref, gir_s, giz_s, gin_s):
    T = N_NODES // NL
    fr = fr_ref[...] * scale_ref[0, 0]
    gi = fr @ Wih_ref[...] + bih_ref[...]
    gir_s[...] = gi[:, 0:HID]
    giz_s[...] = gi[:, HID:2 * HID]
    gin_s[...] = gi[:, 2 * HID:3 * HID]
    Whh = Whh_ref[...]
    Whr = Whh[:, 0:HID]
    Whz = Whh[:, HID:2 * HID]
    Whn = Whh[:, 2 * HID:3 * HID]
    bhh = bhh_ref[...]
    bhr = bhh[:, 0:HID]
    bhz = bhh[:, HID:2 * HID]
    bhn = bhh[:, 2 * HID:3 * HID]

    def one(t, h):
        r = jax.nn.sigmoid(gir_s[pl.ds(t, 1), :] + h @ Whr + bhr)
        z = jax.nn.sigmoid(giz_s[pl.ds(t, 1), :] + h @ Whz + bhz)
        n = jnp.tanh(gin_s[pl.ds(t, 1), :] + r * (h @ Whn + bhn))
        return (1.0 - z) * n + z * h

    def step(i, h):
        for u in range(4):
            h = one(4 * i + u, h)
        return h

    h = lax.fori_loop(0, T // 4, step, jnp.zeros((1, HID), jnp.float32))
    out_ref[...] = (jax.nn.relu(h @ Wc1_ref[...] + bc1_ref[...])
                    @ Wc2_ref[...] + bc2_ref[...])


def _stage_g(fr, scale, Wih, bih, Whh, bhh, Wc1, bc1, Wc2, bc2):
    return pl.pallas_call(
        _tc_g,
        out_shape=jax.ShapeDtypeStruct((1, NC), jnp.float32),
        scratch_shapes=[pltpu.VMEM((N_NODES // NL, HID), jnp.float32)] * 3,
    )(fr, scale, Wih, bih, Whh, bhh, Wc1, bc1, Wc2, bc2)


def kernel(x, edge_index, batch, num_landmarks, W1, b1, W2, b2, Wm, bm,
           Wih, Whh, bih, bhh, Wc1, bc1, Wc2, bc2):
    E = edge_index.shape[1]
    src = jnp.concatenate(
        [edge_index[0], jnp.zeros((E_PAD - E,), jnp.int32)])
    dst = jnp.concatenate(
        [edge_index[1], jnp.full((E_PAD - E,), N_PAD, jnp.int32)])
    x_pad = jnp.pad(x, ((0, N_PAD - x.shape[0]), (0, 0)))

    Z = jnp.zeros((HID, HID), jnp.float32)
    Zi = jnp.zeros((IN, HID), jnp.float32)
    W1p = jnp.concatenate([jnp.concatenate([W1, Zi], 1),
                           jnp.concatenate([Zi, W1], 1)], 0)
    W2p = jnp.concatenate([jnp.concatenate([W2, Z], 1),
                           jnp.concatenate([Z, W2], 1)], 0)
    Wmp = jnp.concatenate([jnp.concatenate([Wm, Z], 1),
                           jnp.concatenate([Z, Wm], 1)], 0)
    E = jnp.repeat(jnp.eye(2, dtype=jnp.float32), HID, axis=1)
    b1p = jnp.concatenate([b1, b1]).reshape(1, 2 * HID)
    b2p = jnp.concatenate([b2, b2]).reshape(1, 2 * HID)
    bmp = jnp.concatenate([bm, bm]).reshape(1, 2 * HID)

    x_p = x_pad.reshape(NP2, 2 * IN)
    degparts = _deg_kernel(dst)
    deg2 = (degparts[0] + degparts[1]).reshape(-1)[:N_PAD].reshape(NP2, 2)
    hs1_p = _stage_b(x_p, deg2, W1p, E)
    acc1 = _conv_kernel(hs1_p.reshape(N_PAD, HID), src, dst)
    hs2_p = _stage_d(acc1.reshape(NP2, 2 * HID), hs1_p, deg2, E, b1p, W2p)
    acc2 = _conv_kernel(hs2_p.reshape(N_PAD, HID), src, dst)
    fr = _stage_f(acc2.reshape(NP2, 2 * HID), hs2_p, deg2, E, b2p, Wmp, bmp)
    scale = (jnp.asarray(num_landmarks).astype(jnp.float32) / NL).reshape(1, 1)
    logits = _stage_g(fr, scale, Wih, bih.reshape(1, 3 * HID), Whh,
                      bhh.reshape(1, 3 * HID), Wc1, bc1.reshape(1, HID // 2),
                      Wc2, bc2.reshape(1, NC))
    return logits
